# baseline (speedup 1.0000x reference)
import functools

import jax
import jax.numpy as jnp
from jax import lax
from jax.experimental import pallas as pl
from jax.experimental.pallas import tpu as pltpu
from jax.experimental.pallas import tpu_sc as plsc

VOCAB = 1000000
MAXLEN = 200
EMBDIM = 64
BATCH = 4096
SEQ = 200

NUM_CORES = 2
NUM_SUBCORES = 16
NUM_WORKERS = NUM_CORES * NUM_SUBCORES
SEQ_PER_WORKER = BATCH // NUM_WORKERS
ROWS_PER_SEQ = SEQ


def _tokposemb_body(x_hbm, tok_hbm, pos_hbm, out_hbm, pos_sh, buf_v, idx_v, sem):
    sid = lax.axis_index("s")
    wid = sid * NUM_CORES + lax.axis_index("c")
    row_base = wid * SEQ_PER_WORKER * ROWS_PER_SEQ

    @pl.when(sid == 0)
    def _():
        pltpu.sync_copy(pos_hbm, buf_v.at[pl.ds(0, MAXLEN)])
        pltpu.sync_copy(buf_v.at[pl.ds(0, MAXLEN)], pos_sh)

    plsc.subcore_barrier()

    def seq_step(i, carry):
        base = row_base + i * ROWS_PER_SEQ
        pltpu.sync_copy(x_hbm.at[pl.ds(base, ROWS_PER_SEQ)], idx_v)
        pltpu.sync_copy(pos_sh, buf_v)
        pltpu.async_copy(tok_hbm.at[idx_v], buf_v, sem, add=True).wait()
        pltpu.sync_copy(buf_v, out_hbm.at[pl.ds(base, ROWS_PER_SEQ)])
        return carry

    lax.fori_loop(0, SEQ_PER_WORKER, seq_step, 0)


@jax.jit
def _tokposemb(x_flat, tok_table, pos_table):
    mesh = plsc.VectorSubcoreMesh(core_axis_name="c", subcore_axis_name="s")
    return pl.kernel(
        _tokposemb_body,
        out_type=jax.ShapeDtypeStruct((BATCH * SEQ, EMBDIM), jnp.float32),
        mesh=mesh,
        scratch_types=[
            pltpu.VMEM_SHARED((MAXLEN, EMBDIM), jnp.float32),
            pltpu.VMEM((ROWS_PER_SEQ, EMBDIM), jnp.float32),
            pltpu.VMEM((ROWS_PER_SEQ,), jnp.int32),
            pltpu.SemaphoreType.DMA,
        ],
        compiler_params=pltpu.CompilerParams(use_tc_tiling_on_sc=False),
    )(x_flat, tok_table, pos_table)


def kernel(x, tok_table, pos_table):
    x_flat = x.reshape(-1).astype(jnp.int32)
    out = _tokposemb(x_flat, tok_table, pos_table)
    return out.reshape(BATCH, SEQ, EMBDIM)


# speedup vs baseline: 1.1799x; 1.1799x over previous
import jax
import jax.numpy as jnp
from jax import lax
from jax.experimental import pallas as pl
from jax.experimental.pallas import tpu as pltpu
from jax.experimental.pallas import tpu_sc as plsc

VOCAB = 1000000
MAXLEN = 200
EMBDIM = 64
BATCH = 4096
SEQ = 200

NUM_CORES = 2
NUM_SUBCORES = 16
NUM_WORKERS = NUM_CORES * NUM_SUBCORES
SEQ_PER_WORKER = BATCH // NUM_WORKERS
CS = 2
ROWS_PER_CHUNK = CS * SEQ
NCHUNK = SEQ_PER_WORKER // CS
NBUF = 3
ROWS_PER_WORKER = SEQ_PER_WORKER * SEQ


def _tokposemb_body(x_hbm, tok_hbm, pos_hbm, out_hbm,
                    pos_sh, idx_v, buf_v, sem_p, sem_g, sem_w):
    sid = lax.axis_index("s")
    wid = sid * NUM_CORES + lax.axis_index("c")
    row_base = wid * ROWS_PER_WORKER

    pltpu.sync_copy(x_hbm.at[pl.ds(row_base, ROWS_PER_WORKER)], idx_v)

    @pl.when(sid == 0)
    def _():
        pltpu.sync_copy(pos_hbm, buf_v.at[0, pl.ds(0, MAXLEN)])
        for c in range(CS):
            pltpu.sync_copy(buf_v.at[0, pl.ds(0, MAXLEN)],
                            pos_sh.at[pl.ds(c * MAXLEN, MAXLEN)])

    plsc.subcore_barrier()

    def chunk_rows(i):
        return row_base + i * ROWS_PER_CHUNK

    def issue_posfill(b):
        pltpu.async_copy(pos_sh, buf_v.at[b], sem_p.at[b])

    def wait_posfill(b):
        pltpu.make_async_copy(pos_sh, buf_v.at[b], sem_p.at[b]).wait()

    def issue_gather(i, b):
        pltpu.async_copy(
            tok_hbm.at[idx_v.at[pl.ds(i * ROWS_PER_CHUNK, ROWS_PER_CHUNK)]],
            buf_v.at[b], sem_g.at[b], add=True)

    def wait_gather(i, b):
        pltpu.make_async_copy(
            tok_hbm.at[idx_v.at[pl.ds(i * ROWS_PER_CHUNK, ROWS_PER_CHUNK)]],
            buf_v.at[b], sem_g.at[b]).wait()

    def issue_wb(i, b):
        pltpu.async_copy(buf_v.at[b],
                         out_hbm.at[pl.ds(chunk_rows(i), ROWS_PER_CHUNK)],
                         sem_w.at[b])

    def wait_wb(i, b):
        pltpu.make_async_copy(buf_v.at[b],
                              out_hbm.at[pl.ds(chunk_rows(i), ROWS_PER_CHUNK)],
                              sem_w.at[b]).wait()

    issue_posfill(0)

    n_outer = (NCHUNK + 1 + NBUF - 1) // NBUF

    def outer(g, carry):
        for b_off in range(NBUF):
            i = g * NBUF + b_off
            b = b_off

            @pl.when(i < NCHUNK)
            def _():
                wait_posfill(b)
                issue_gather(i, b)

            bn = (b_off + 1) % NBUF

            @pl.when(i + 1 < NCHUNK)
            def _():
                @pl.when(i + 1 >= NBUF)
                def _():
                    wait_wb(i + 1 - NBUF, bn)
                issue_posfill(bn)

            bp = (b_off - 1) % NBUF

            @pl.when(jnp.logical_and(i >= 1, i <= NCHUNK))
            def _():
                wait_gather(i - 1, bp)
                issue_wb(i - 1, bp)
        return carry

    lax.fori_loop(0, n_outer, outer, 0)

    for j in range(NCHUNK - NBUF, NCHUNK):
        if j >= 0:
            wait_wb(j, j % NBUF)


@jax.jit
def _tokposemb(x_flat, tok_table, pos_table):
    mesh = plsc.VectorSubcoreMesh(core_axis_name="c", subcore_axis_name="s")
    return pl.kernel(
        _tokposemb_body,
        out_type=jax.ShapeDtypeStruct((BATCH * SEQ, EMBDIM), jnp.float32),
        mesh=mesh,
        scratch_types=[
            pltpu.VMEM_SHARED((ROWS_PER_CHUNK, EMBDIM), jnp.float32),
            pltpu.VMEM((ROWS_PER_WORKER,), jnp.int32),
            pltpu.VMEM((NBUF, ROWS_PER_CHUNK, EMBDIM), jnp.float32),
            pltpu.SemaphoreType.DMA((NBUF,)),
            pltpu.SemaphoreType.DMA((NBUF,)),
            pltpu.SemaphoreType.DMA((NBUF,)),
        ],
        compiler_params=pltpu.CompilerParams(use_tc_tiling_on_sc=False),
    )(x_flat, tok_table, pos_table)


def kernel(x, tok_table, pos_table):
    x_flat = x.reshape(-1).astype(jnp.int32)
    out = _tokposemb(x_flat, tok_table, pos_table)
    return out.reshape(BATCH, SEQ, EMBDIM)
